# Initial kernel scaffold; baseline (speedup 1.0000x reference)
import jax
import jax.numpy as jnp
from jax.experimental import pallas as pl


def kernel(words, entity_id, batch_feature_bert, edge_src, edge_dst, h_t_pairs, rel_W, loop_W, h_bias, W1, b1, W2, b2):
    raise NotImplementedError("write your pallas kernel here")


# speedup vs baseline: 8.9098x; 8.9098x over previous
import functools

import jax
import jax.numpy as jnp
from jax import lax
from jax.experimental import pallas as pl
from jax.experimental.pallas import tpu as pltpu, tpu_sc as plsc

BSZ = 4
N = 96
D = 808
R = 5
L = 2
E = 2048
HT = 128
BANK = D * (L + 1)
H1 = 2 * BANK
OUT = 97
NPAIR = BSZ * R
ASZ = N * N


_PAIRS_PER_CORE = NPAIR // 2


def _adj_sc_body(src_hbm, dst_hbm, out_hbm, src_v, dst_v, idx_v, ones_v, abuf, shared):
    c = lax.axis_index("c")
    s = lax.axis_index("s")
    pair = c * _PAIRS_PER_CORE + s

    @pl.when(s < _PAIRS_PER_CORE)
    def _():
        pltpu.sync_copy(src_hbm.at[pair], src_v)
        pltpu.sync_copy(dst_hbm.at[pair], dst_v)

        def zero_body(i, _):
            abuf[pl.ds(i * 16, 16)] = jnp.zeros((16,), jnp.float32)
            return 0

        lax.fori_loop(0, ASZ // 16, zero_body, 0)

        def ones_body(i, _):
            ones_v[pl.ds(i * 16, 16)] = jnp.full((16,), 1.0, jnp.float32)
            return 0

        lax.fori_loop(0, 8, ones_body, 0)

        base = s * ASZ

        def idx_body(i, _):
            j = i // 8
            k = (i % 8) * 16
            sv = src_v[j, pl.ds(k, 16)]
            dv = dst_v[j, pl.ds(k, 16)]
            idx_v[j, pl.ds(k, 16)] = dv * N + sv + base
            return 0

        lax.fori_loop(0, E // 16, idx_body, 0)

        pltpu.sync_copy(abuf, shared.at[pl.ds(s * ASZ, ASZ)])
        for j in range(16):
            pltpu.sync_copy(ones_v, shared.at[idx_v.at[j]], add=True)

        pltpu.sync_copy(shared.at[pl.ds(s * ASZ, ASZ)], abuf)
        pltpu.sync_copy(abuf, out_hbm.at[pair])


def _build_adjacency(edge_src, edge_dst):
    src3 = edge_src.reshape(NPAIR, 16, E // 16).astype(jnp.int32)
    dst3 = edge_dst.reshape(NPAIR, 16, E // 16).astype(jnp.int32)
    mesh = plsc.VectorSubcoreMesh(core_axis_name="c", subcore_axis_name="s")
    f = pl.kernel(
        _adj_sc_body,
        out_type=jax.ShapeDtypeStruct((NPAIR, ASZ), jnp.float32),
        mesh=mesh,
        scratch_types=[
            pltpu.VMEM((16, E // 16), jnp.int32),
            pltpu.VMEM((16, E // 16), jnp.int32),
            pltpu.VMEM((16, E // 16), jnp.int32),
            pltpu.VMEM((E // 16,), jnp.float32),
            pltpu.VMEM((ASZ,), jnp.float32),
            pltpu.VMEM_SHARED((_PAIRS_PER_CORE * ASZ,), jnp.float32),
        ],
    )
    return f(src3, dst3)


def _rgcn_body(x_ref, w_ref, a_ref, b_ref, out_ref, acc_ref):
    m = pl.program_id(0)
    r = m % (R + 1)
    h = jnp.dot(x_ref[...], w_ref[0], preferred_element_type=jnp.float32)

    @pl.when(r == 0)
    def _():
        acc_ref[...] = jnp.zeros_like(acc_ref)

    @pl.when(r < R)
    def _():
        for b in range(BSZ):
            ab = a_ref[0, b]
            deg = jnp.sum(ab, axis=1)
            recip = 1.0 / jnp.maximum(deg, 1.0)
            hb = h[b * N:(b + 1) * N, :]
            p = jnp.dot(ab, hb, preferred_element_type=jnp.float32)
            acc_ref[b * N:(b + 1) * N, :] += p * recip[:, None]

    @pl.when(r == R)
    def _():
        out_ref[0] = jnp.maximum(acc_ref[...] + h + b_ref[0, 0:1, :], 0.0)


def _rgcn(x, w_all, a, bias8):
    grid = (L * (R + 1),)
    return pl.pallas_call(
        _rgcn_body,
        grid=grid,
        in_specs=[
            pl.BlockSpec((BSZ * N, D), lambda m: (0, 0)),
            pl.BlockSpec((1, D, D), lambda m: (m, 0, 0)),
            pl.BlockSpec((1, BSZ, N, N), lambda m: (jnp.minimum(m % (R + 1), R - 1), 0, 0, 0)),
            pl.BlockSpec((1, 8, D), lambda m: (m // (R + 1), 0, 0)),
        ],
        out_specs=pl.BlockSpec((1, BSZ * N, D), lambda m: (m // (R + 1), 0, 0)),
        out_shape=jax.ShapeDtypeStruct((L, BSZ * N, D), jnp.float32),
        scratch_shapes=[pltpu.VMEM((BSZ * N, D), jnp.float32)],
    )(x, w_all, a, bias8)


def _feat_body(bank_ref, hi_ref, ti_ref, out_ref, selh_ref, selt_ref):
    j = pl.program_id(0)

    @pl.when(j == 0)
    def _():
        gi = lax.broadcasted_iota(jnp.int32, (BSZ * HT, BSZ * N), 1)
        selh_ref[...] = (gi == hi_ref[:, 0:1]).astype(jnp.float32)
        selt_ref[...] = (gi == ti_ref[:, 0:1]).astype(jnp.float32)

    bank = bank_ref[0]
    h = jnp.dot(selh_ref[...], bank, preferred_element_type=jnp.float32)
    t = jnp.dot(selt_ref[...], bank, preferred_element_type=jnp.float32)
    out_ref[0] = h
    out_ref[1] = t
    out_ref[2] = jnp.abs(h - t)
    out_ref[3] = h * t


def _feat(bank3, hidx, tidx):
    grid = (L + 1,)
    return pl.pallas_call(
        _feat_body,
        grid=grid,
        in_specs=[
            pl.BlockSpec((1, BSZ * N, D), lambda j: (j, 0, 0)),
            pl.BlockSpec((BSZ * HT, 128), lambda j: (0, 0)),
            pl.BlockSpec((BSZ * HT, 128), lambda j: (0, 0)),
        ],
        out_specs=pl.BlockSpec((4, BSZ * HT, D), lambda j: (j, 0, 0)),
        out_shape=jax.ShapeDtypeStruct((4 * (L + 1), BSZ * HT, D), jnp.float32),
        scratch_shapes=[
            pltpu.VMEM((BSZ * HT, BSZ * N), jnp.float32),
            pltpu.VMEM((BSZ * HT, BSZ * N), jnp.float32),
        ],
    )(bank3, hidx, tidx)


_KT = 4 * (L + 1)


def _mlp_body(f_ref, w1_ref, b1_ref, w2_ref, b2_ref, out_ref, hacc_ref):
    m = pl.program_id(0)
    part = jnp.dot(f_ref[0], w1_ref[0], preferred_element_type=jnp.float32)

    @pl.when(m == 0)
    def _():
        hacc_ref[...] = part

    @pl.when(m > 0)
    def _():
        hacc_ref[...] += part

    @pl.when(m == _KT - 1)
    def _():
        h = jnp.maximum(hacc_ref[...] + b1_ref[0:1, :], 0.0)
        out_ref[...] = jnp.dot(h, w2_ref[...],
                               preferred_element_type=jnp.float32) + b2_ref[0:1, :]


def _mlp(feat, w1p, b1x, w2, b2x):
    grid = (_KT,)
    return pl.pallas_call(
        _mlp_body,
        grid=grid,
        in_specs=[
            pl.BlockSpec((1, BSZ * HT, D), lambda m: (m, 0, 0)),
            pl.BlockSpec((1, D, H1), lambda m: (m, 0, 0)),
            pl.BlockSpec((8, H1), lambda m: (0, 0)),
            pl.BlockSpec((H1, OUT), lambda m: (0, 0)),
            pl.BlockSpec((8, OUT), lambda m: (0, 0)),
        ],
        out_specs=pl.BlockSpec((BSZ * HT, OUT), lambda m: (0, 0)),
        out_shape=jax.ShapeDtypeStruct((BSZ * HT, OUT), jnp.float32),
        scratch_shapes=[pltpu.VMEM((BSZ * HT, H1), jnp.float32)],
    )(feat, w1p, b1x, w2, b2x)


def kernel(words, entity_id, batch_feature_bert, edge_src, edge_dst, h_t_pairs,
           rel_W, loop_W, h_bias, W1, b1, W2, b2):
    x = batch_feature_bert.reshape(BSZ * N, D)

    a = _build_adjacency(edge_src, edge_dst)
    a = a.reshape(BSZ, R, N, N).transpose(1, 0, 2, 3)

    w_all = jnp.concatenate([rel_W, loop_W[:, None]], axis=1).reshape(L * (R + 1), D, D)
    bias8 = jnp.broadcast_to(h_bias[:, None, :], (L, 8, D))

    out = _rgcn(x, w_all, a, bias8)
    bank3 = jnp.concatenate([x[None], out], axis=0)

    p = h_t_pairs + (h_t_pairs == 0).astype(h_t_pairs.dtype) - 1
    g = p.astype(jnp.int32) + (jnp.arange(BSZ, dtype=jnp.int32) * N)[:, None, None]
    hidx = jnp.broadcast_to(g[:, :, 0].reshape(BSZ * HT, 1), (BSZ * HT, 128))
    tidx = jnp.broadcast_to(g[:, :, 1].reshape(BSZ * HT, 1), (BSZ * HT, 128))

    feat = _feat(bank3, hidx, tidx)

    w1p = W1.reshape(4, L + 1, D, H1).transpose(1, 0, 2, 3).reshape(_KT, D, H1)
    b1x = jnp.broadcast_to(b1[None, :], (8, H1))
    b2x = jnp.broadcast_to(b2[None, :], (8, OUT))

    res = _mlp(feat, w1p, b1x, W2, b2x)
    return res.reshape(BSZ, HT, OUT)


# speedup vs baseline: 23.4428x; 2.6311x over previous
import jax
import jax.numpy as jnp
from jax import lax
from jax.experimental import pallas as pl
from jax.experimental.pallas import tpu as pltpu, tpu_sc as plsc

BSZ = 4
N = 96
D = 808
R = 5
L = 2
E = 2048
HT = 128
BANK = D * (L + 1)
H1 = 2 * BANK
OUT = 97
NPAIR = BSZ * R
ASZ = N * N
BF = jnp.bfloat16


_PAIRS_PER_CORE = NPAIR // 2


def _adj_sc_body(src_hbm, dst_hbm, out_hbm, src_v, dst_v, idx_v, ones_v, abuf, shared):
    c = lax.axis_index("c")
    s = lax.axis_index("s")
    pair = c * _PAIRS_PER_CORE + s
    r = pair // BSZ
    b = pair % BSZ

    @pl.when(s < _PAIRS_PER_CORE)
    def _():
        pltpu.sync_copy(src_hbm.at[b, r], src_v)
        pltpu.sync_copy(dst_hbm.at[b, r], dst_v)

        def zero_body(i, _):
            abuf[pl.ds(i * 16, 16)] = jnp.zeros((16,), jnp.float32)
            return 0

        lax.fori_loop(0, ASZ // 16, zero_body, 0)

        def ones_body(i, _):
            ones_v[pl.ds(i * 16, 16)] = jnp.full((16,), 1.0, jnp.float32)
            return 0

        lax.fori_loop(0, 8, ones_body, 0)

        base = s * ASZ

        def idx_body(i, _):
            j = i // 8
            k = (i % 8) * 16
            sv = src_v[j, pl.ds(k, 16)]
            dv = dst_v[j, pl.ds(k, 16)]
            idx_v[j, pl.ds(k, 16)] = dv * N + sv + base
            return 0

        lax.fori_loop(0, E // 16, idx_body, 0)

        pltpu.sync_copy(abuf, shared.at[pl.ds(s * ASZ, ASZ)])
        for j in range(16):
            pltpu.sync_copy(ones_v, shared.at[idx_v.at[j]], add=True)

        pltpu.sync_copy(shared.at[pl.ds(s * ASZ, ASZ)], abuf)
        pltpu.sync_copy(abuf, out_hbm.at[pair])


def _build_adjacency(edge_src, edge_dst):
    src4 = edge_src.reshape(BSZ, R, 16, E // 16).astype(jnp.int32)
    dst4 = edge_dst.reshape(BSZ, R, 16, E // 16).astype(jnp.int32)
    mesh = plsc.VectorSubcoreMesh(core_axis_name="c", subcore_axis_name="s")
    f = pl.kernel(
        _adj_sc_body,
        out_type=jax.ShapeDtypeStruct((NPAIR, ASZ), jnp.float32),
        mesh=mesh,
        scratch_types=[
            pltpu.VMEM((16, E // 16), jnp.int32),
            pltpu.VMEM((16, E // 16), jnp.int32),
            pltpu.VMEM((16, E // 16), jnp.int32),
            pltpu.VMEM((E // 16,), jnp.float32),
            pltpu.VMEM((ASZ,), jnp.float32),
            pltpu.VMEM_SHARED((_PAIRS_PER_CORE * ASZ,), jnp.float32),
        ],
    )
    return f(src4, dst4)


def _rgcn_body(x_ref, rw_ref, lw_ref, a_ref, b_ref, out_ref, acc_ref):
    m = pl.program_id(0)
    r = m % (R + 1)
    xb = x_ref[...].astype(BF)

    @pl.when(r == 0)
    def _():
        acc_ref[...] = jnp.zeros_like(acc_ref)

    @pl.when(r < R)
    def _():
        h = jnp.dot(xb, rw_ref[0, 0].astype(BF), preferred_element_type=jnp.float32)
        for b in range(BSZ):
            ab = a_ref[0, b]
            deg = jnp.sum(ab, axis=1)
            recip = 1.0 / jnp.maximum(deg, 1.0)
            hb = h[b * N:(b + 1) * N, :]
            p = jnp.dot(ab, hb, preferred_element_type=jnp.float32)
            acc_ref[b * N:(b + 1) * N, :] += p * recip[:, None]

    @pl.when(r == R)
    def _():
        h = jnp.dot(xb, lw_ref[0].astype(BF), preferred_element_type=jnp.float32)
        out_ref[0] = jnp.maximum(acc_ref[...] + h + b_ref[0, 0:1, :], 0.0)


def _rgcn(x, rel_W, loop_W, a, bias8):
    grid = (L * (R + 1),)
    return pl.pallas_call(
        _rgcn_body,
        grid=grid,
        in_specs=[
            pl.BlockSpec((BSZ * N, D), lambda m: (0, 0)),
            pl.BlockSpec((1, 1, D, D),
                         lambda m: (m // (R + 1), jnp.minimum(m % (R + 1), R - 1), 0, 0)),
            pl.BlockSpec((1, D, D), lambda m: (m // (R + 1), 0, 0)),
            pl.BlockSpec((1, BSZ, N, N),
                         lambda m: (jnp.minimum(m % (R + 1), R - 1), 0, 0, 0)),
            pl.BlockSpec((1, 8, D), lambda m: (m // (R + 1), 0, 0)),
        ],
        out_specs=pl.BlockSpec((1, BSZ * N, D), lambda m: (m // (R + 1), 0, 0)),
        out_shape=jax.ShapeDtypeStruct((L, BSZ * N, D), jnp.float32),
        scratch_shapes=[pltpu.VMEM((BSZ * N, D), jnp.float32)],
    )(x, rel_W, loop_W, a, bias8)


_KT = 4 * (L + 1)


def _fmlp_body(bank_ref, hi_ref, ti_ref, w1_ref, b1_ref, w2_ref, b2_ref, out_ref,
               selh, selt, hbuf, tbuf, ftile, hacc):
    m = pl.program_id(0)
    kind = m // (L + 1)
    j = m % (L + 1)

    @pl.when(m == 0)
    def _():
        gi = lax.broadcasted_iota(jnp.int32, (BSZ * HT, BSZ * N), 1)
        selh[...] = (gi == hi_ref[:, 0:1]).astype(BF)
        selt[...] = (gi == ti_ref[:, 0:1]).astype(BF)

    @pl.when(kind == 0)
    def _():
        bb = bank_ref[0].astype(BF)
        hv = jnp.dot(selh[...], bb, preferred_element_type=jnp.float32)
        tv = jnp.dot(selt[...], bb, preferred_element_type=jnp.float32)
        hbuf[j] = hv.astype(BF)
        tbuf[j] = tv.astype(BF)
        ftile[...] = hv.astype(BF)

    @pl.when(kind == 1)
    def _():
        ftile[...] = tbuf[j]

    @pl.when(kind == 2)
    def _():
        hv = hbuf[j].astype(jnp.float32)
        tv = tbuf[j].astype(jnp.float32)
        ftile[...] = jnp.abs(hv - tv).astype(BF)

    @pl.when(kind == 3)
    def _():
        hv = hbuf[j].astype(jnp.float32)
        tv = tbuf[j].astype(jnp.float32)
        ftile[...] = (hv * tv).astype(BF)

    part = jnp.dot(ftile[...], w1_ref[0].astype(BF),
                   preferred_element_type=jnp.float32)

    @pl.when(m == 0)
    def _():
        hacc[...] = part

    @pl.when(m > 0)
    def _():
        hacc[...] += part

    @pl.when(m == _KT - 1)
    def _():
        h = jnp.maximum(hacc[...] + b1_ref[0:1, :], 0.0).astype(BF)
        out_ref[...] = jnp.dot(h, w2_ref[...].astype(BF),
                               preferred_element_type=jnp.float32) + b2_ref[0:1, :]


def _fmlp(bank3, hidx, tidx, w1r, b1x, w2, b2x):
    grid = (_KT,)
    return pl.pallas_call(
        _fmlp_body,
        grid=grid,
        in_specs=[
            pl.BlockSpec((1, BSZ * N, D), lambda m: (jnp.minimum(m, L), 0, 0)),
            pl.BlockSpec((BSZ * HT, 128), lambda m: (0, 0)),
            pl.BlockSpec((BSZ * HT, 128), lambda m: (0, 0)),
            pl.BlockSpec((1, D, H1), lambda m: (m, 0, 0)),
            pl.BlockSpec((8, H1), lambda m: (0, 0)),
            pl.BlockSpec((H1, OUT), lambda m: (0, 0)),
            pl.BlockSpec((8, OUT), lambda m: (0, 0)),
        ],
        out_specs=pl.BlockSpec((BSZ * HT, OUT), lambda m: (0, 0)),
        out_shape=jax.ShapeDtypeStruct((BSZ * HT, OUT), jnp.float32),
        compiler_params=pltpu.CompilerParams(vmem_limit_bytes=100 * 1024 * 1024),
        scratch_shapes=[
            pltpu.VMEM((BSZ * HT, BSZ * N), BF),
            pltpu.VMEM((BSZ * HT, BSZ * N), BF),
            pltpu.VMEM((L + 1, BSZ * HT, D), BF),
            pltpu.VMEM((L + 1, BSZ * HT, D), BF),
            pltpu.VMEM((BSZ * HT, D), BF),
            pltpu.VMEM((BSZ * HT, H1), jnp.float32),
        ],
    )(bank3, hidx, tidx, w1r, b1x, w2, b2x)


def kernel(words, entity_id, batch_feature_bert, edge_src, edge_dst, h_t_pairs,
           rel_W, loop_W, h_bias, W1, b1, W2, b2):
    x = batch_feature_bert.reshape(BSZ * N, D)

    a = _build_adjacency(edge_src, edge_dst).reshape(R, BSZ, N, N)

    bias8 = jnp.broadcast_to(h_bias[:, None, :], (L, 8, D))
    out = _rgcn(x, rel_W, loop_W, a, bias8)
    bank3 = jnp.concatenate([x[None], out], axis=0)

    p = h_t_pairs + (h_t_pairs == 0).astype(h_t_pairs.dtype) - 1
    g = p.astype(jnp.int32) + (jnp.arange(BSZ, dtype=jnp.int32) * N)[:, None, None]
    hidx = jnp.broadcast_to(g[:, :, 0].reshape(BSZ * HT, 1), (BSZ * HT, 128))
    tidx = jnp.broadcast_to(g[:, :, 1].reshape(BSZ * HT, 1), (BSZ * HT, 128))

    w1r = W1.reshape(_KT, D, H1)
    b1x = jnp.broadcast_to(b1[None, :], (8, H1))
    b2x = jnp.broadcast_to(b2[None, :], (8, OUT))

    res = _fmlp(bank3, hidx, tidx, w1r, b1x, W2, b2x)
    return res.reshape(BSZ, HT, OUT)
